# baseline (speedup 1.0000x reference)
import functools

import jax
import jax.numpy as jnp
from jax import lax
from jax.experimental import pallas as pl
from jax.experimental.pallas import tpu as pltpu
from jax.experimental.pallas import tpu_sc as plsc

VOCAB = 100000
DIM = 128
BATCH = 4096
CTX = 20

NC = 2
NS = 16
NW = NC * NS
B_PER_W = BATCH // NW
CH = 32
NCH = B_PER_W // CH
LANES = 16

VB = 800
NVT = VOCAB // VB


def _sc_avg_body(idx_hbm, table_hbm, out_hbm, idx_v, rows_v, acc_v, sem):
    wid = lax.axis_index("s") * NC + lax.axis_index("c")
    base = wid * B_PER_W

    def chunk_body(ch, carry):
        pltpu.sync_copy(idx_hbm.at[wid, ch], idx_v)
        handles = [
            pltpu.async_copy(table_hbm.at[idx_v.at[t]], rows_v.at[t], sem)
            for t in range(CTX)
        ]
        for h in handles:
            h.wait()

        def row_body(r, c2):
            for d in range(DIM // LANES):
                sl = pl.ds(d * LANES, LANES)
                a = rows_v[0, r, sl]
                for t in range(1, CTX):
                    a = a + rows_v[t, r, sl]
                acc_v[r, sl] = a * (1.0 / CTX)
            return c2

        lax.fori_loop(0, CH, row_body, 0)
        pltpu.sync_copy(acc_v, out_hbm.at[pl.ds(base + ch * CH, CH)])
        return carry

    lax.fori_loop(0, NCH, chunk_body, 0)


_sc_avg = functools.partial(
    pl.kernel,
    mesh=plsc.VectorSubcoreMesh(core_axis_name="c", subcore_axis_name="s"),
    out_type=jax.ShapeDtypeStruct((BATCH, DIM), jnp.float32),
    scratch_types=[
        pltpu.VMEM((CTX, CH), jnp.int32),
        pltpu.VMEM((CTX, CH, DIM), jnp.float32),
        pltpu.VMEM((CH, DIM), jnp.float32),
        pltpu.SemaphoreType.DMA,
    ],
)(_sc_avg_body)


def _tc_body(wt_ref, b_ref, avgt_ref, out_ref, s_ref, lprev_ref):
    p = pl.program_id(0)
    v = pl.program_id(1)

    @pl.when((p == 0) & (v == 0))
    def _init():
        s_ref[...] = jnp.zeros_like(s_ref)

    logits = lax.dot_general(
        wt_ref[...], avgt_ref[...], (((1,), (0,)), ((), ())),
        preferred_element_type=jnp.float32,
    ) + b_ref[...]

    @pl.when(p == 0)
    def _acc():
        @pl.when(v > 0)
        def _drain():
            s_ref[...] += jnp.sum(jnp.exp(lprev_ref[...].astype(jnp.float32)), axis=0,
                                  keepdims=True)
        lprev_ref[...] = logits.astype(jnp.bfloat16)

    @pl.when(p == 1)
    def _write():
        @pl.when(v == 0)
        def _finalize():
            s = s_ref[...] + jnp.sum(jnp.exp(lprev_ref[...].astype(jnp.float32)), axis=0,
                                     keepdims=True)
            s_ref[...] = jnp.log(s)
        out_ref[...] = logits - s_ref[...]


def kernel(context_words, emb_table, W, b):
    idx4 = (
        context_words.astype(jnp.int32)
        .reshape(NW, NCH, CH, CTX)
        .transpose(0, 1, 3, 2)
    )
    avg = _sc_avg(idx4, emb_table)

    wt16 = W.T.astype(jnp.bfloat16)
    avgt16 = avg.T.astype(jnp.bfloat16)
    b2 = b.reshape(VOCAB, 1)
    out_t = pl.pallas_call(
        _tc_body,
        grid=(2, NVT),
        in_specs=[
            pl.BlockSpec((VB, DIM), lambda p, v: (v, 0)),
            pl.BlockSpec((VB, 1), lambda p, v: (v, 0)),
            pl.BlockSpec((DIM, BATCH), lambda p, v: (0, 0)),
        ],
        out_specs=pl.BlockSpec((VB, BATCH), lambda p, v: (v * p, 0)),
        out_shape=jax.ShapeDtypeStruct((VOCAB, BATCH), jnp.float32),
        scratch_shapes=[
            pltpu.VMEM((1, BATCH), jnp.float32),
            pltpu.VMEM((VB, BATCH), jnp.bfloat16),
        ],
        compiler_params=pltpu.CompilerParams(
            dimension_semantics=("arbitrary", "arbitrary"),
            vmem_limit_bytes=66_000_000,
        ),
    )(wt16, b2, avgt16)
    return out_t.T


# speedup vs baseline: 1.1038x; 1.1038x over previous
import functools

import jax
import jax.numpy as jnp
from jax import lax
from jax.experimental import pallas as pl
from jax.experimental.pallas import tpu as pltpu
from jax.experimental.pallas import tpu_sc as plsc

VOCAB = 100000
DIM = 128
BATCH = 4096
CTX = 20

NC = 2
NS = 16
NW = NC * NS
B_PER_W = BATCH // NW
CH = 32
NCH = B_PER_W // CH
LANES = 16

VB = 800
NVT = VOCAB // VB


def _sc_avg_body(idx_hbm, table_hbm, out_hbm, idx_v, rows_v, acc_v, sem):
    wid = lax.axis_index("s") * NC + lax.axis_index("c")
    base = wid * B_PER_W

    def chunk_body(ch, carry):
        pltpu.sync_copy(idx_hbm.at[wid, ch], idx_v)
        handles = [
            pltpu.async_copy(table_hbm.at[idx_v.at[t]], rows_v.at[t], sem)
            for t in range(CTX)
        ]
        for h in handles:
            h.wait()

        def row_body(r, c2):
            for d in range(DIM // LANES):
                sl = pl.ds(d * LANES, LANES)
                a = rows_v[0, r, sl]
                for t in range(1, CTX):
                    a = a + rows_v[t, r, sl]
                acc_v[r, sl] = a * (1.0 / CTX)
            return c2

        lax.fori_loop(0, CH, row_body, 0)
        pltpu.sync_copy(acc_v, out_hbm.at[pl.ds(base + ch * CH, CH)])
        return carry

    lax.fori_loop(0, NCH, chunk_body, 0)


_sc_avg = functools.partial(
    pl.kernel,
    mesh=plsc.VectorSubcoreMesh(core_axis_name="c", subcore_axis_name="s"),
    out_type=jax.ShapeDtypeStruct((BATCH, DIM), jnp.float32),
    scratch_types=[
        pltpu.VMEM((CTX, CH), jnp.int32),
        pltpu.VMEM((CTX, CH, DIM), jnp.float32),
        pltpu.VMEM((CH, DIM), jnp.float32),
        pltpu.SemaphoreType.DMA,
    ],
)(_sc_avg_body)


def _tc_body(wt_ref, b_ref, avgt_ref, out_ref, s_ref):
    p = pl.program_id(0)
    v = pl.program_id(1)

    @pl.when((p == 0) & (v == 0))
    def _init():
        s_ref[...] = jnp.zeros_like(s_ref)

    logits = lax.dot_general(
        wt_ref[...], avgt_ref[...], (((1,), (0,)), ((), ())),
        preferred_element_type=jnp.float32,
    ) + b_ref[...]

    @pl.when(p == 0)
    def _acc():
        e16 = jnp.exp(logits).astype(jnp.bfloat16)
        ones = jnp.ones((1, VB), jnp.bfloat16)
        s_ref[...] += lax.dot_general(
            ones, e16, (((1,), (0,)), ((), ())),
            preferred_element_type=jnp.float32,
        )

    @pl.when(p == 1)
    def _write():
        @pl.when(v == 0)
        def _finalize():
            s_ref[...] = jnp.log(s_ref[...])
        out_ref[...] = logits - s_ref[...]


def kernel(context_words, emb_table, W, b):
    idx4 = (
        context_words.astype(jnp.int32)
        .reshape(NW, NCH, CH, CTX)
        .transpose(0, 1, 3, 2)
    )
    avg = _sc_avg(idx4, emb_table)

    wt16 = W.T.astype(jnp.bfloat16)
    avgt16 = avg.T.astype(jnp.bfloat16)
    b2 = b.reshape(VOCAB, 1)
    out_t = pl.pallas_call(
        _tc_body,
        grid=(2, NVT),
        in_specs=[
            pl.BlockSpec((VB, DIM), lambda p, v: (v, 0)),
            pl.BlockSpec((VB, 1), lambda p, v: (v, 0)),
            pl.BlockSpec((DIM, BATCH), lambda p, v: (0, 0)),
        ],
        out_specs=pl.BlockSpec((VB, BATCH), lambda p, v: (v * p, 0)),
        out_shape=jax.ShapeDtypeStruct((VOCAB, BATCH), jnp.float32),
        scratch_shapes=[
            pltpu.VMEM((1, BATCH), jnp.float32),
        ],
        compiler_params=pltpu.CompilerParams(
            dimension_semantics=("arbitrary", "arbitrary"),
            vmem_limit_bytes=66_000_000,
        ),
    )(wt16, b2, avgt16)
    return out_t.T


# speedup vs baseline: 1.1042x; 1.0004x over previous
import functools

import jax
import jax.numpy as jnp
from jax import lax
from jax.experimental import pallas as pl
from jax.experimental.pallas import tpu as pltpu
from jax.experimental.pallas import tpu_sc as plsc

VOCAB = 100000
DIM = 128
BATCH = 4096
CTX = 20

NC = 2
NS = 16
NW = NC * NS
B_PER_W = BATCH // NW
CH = 32
NCH = B_PER_W // CH
LANES = 16

VB = 1000
NVT = VOCAB // VB


def _sc_avg_body(idx_hbm, table_hbm, out_hbm, idx_v, rows_v, acc_v, sem):
    wid = lax.axis_index("s") * NC + lax.axis_index("c")
    base = wid * B_PER_W

    def chunk_body(ch, carry):
        pltpu.sync_copy(idx_hbm.at[wid, ch], idx_v)
        handles = [
            pltpu.async_copy(table_hbm.at[idx_v.at[t]], rows_v.at[t], sem)
            for t in range(CTX)
        ]
        for h in handles:
            h.wait()

        def row_body(r, c2):
            for d in range(DIM // LANES):
                sl = pl.ds(d * LANES, LANES)
                a = rows_v[0, r, sl]
                for t in range(1, CTX):
                    a = a + rows_v[t, r, sl]
                acc_v[r, sl] = a * (1.0 / CTX)
            return c2

        lax.fori_loop(0, CH, row_body, 0)
        pltpu.sync_copy(acc_v, out_hbm.at[pl.ds(base + ch * CH, CH)])
        return carry

    lax.fori_loop(0, NCH, chunk_body, 0)


_sc_avg = functools.partial(
    pl.kernel,
    mesh=plsc.VectorSubcoreMesh(core_axis_name="c", subcore_axis_name="s"),
    out_type=jax.ShapeDtypeStruct((BATCH, DIM), jnp.float32),
    scratch_types=[
        pltpu.VMEM((CTX, CH), jnp.int32),
        pltpu.VMEM((CTX, CH, DIM), jnp.float32),
        pltpu.VMEM((CH, DIM), jnp.float32),
        pltpu.SemaphoreType.DMA,
    ],
)(_sc_avg_body)


def _tc_body(wt_ref, b_ref, avgt_ref, out_ref, s_ref):
    p = pl.program_id(0)
    v = pl.program_id(1)

    @pl.when((p == 0) & (v == 0))
    def _init():
        s_ref[...] = jnp.zeros_like(s_ref)

    raw = lax.dot_general(
        wt_ref[...], avgt_ref[...], (((1,), (0,)), ((), ())),
        preferred_element_type=jnp.float32,
    )

    @pl.when(p == 0)
    def _acc():
        e16 = jnp.exp(raw).astype(jnp.bfloat16)
        eb = jnp.exp(b_ref[...]).astype(jnp.bfloat16)
        s_ref[...] += lax.dot_general(
            eb, e16, (((0,), (0,)), ((), ())),
            preferred_element_type=jnp.float32,
        )

    @pl.when(p == 1)
    def _write():
        @pl.when(v == 0)
        def _finalize():
            s_ref[...] = jnp.log(s_ref[...])
        out_ref[...] = (raw + b_ref[...]) - s_ref[...]


def kernel(context_words, emb_table, W, b):
    idx4 = (
        context_words.astype(jnp.int32)
        .reshape(NW, NCH, CH, CTX)
        .transpose(0, 1, 3, 2)
    )
    avg = _sc_avg(idx4, emb_table)

    wt16 = W.T.astype(jnp.bfloat16)
    avgt16 = avg.T.astype(jnp.bfloat16)
    b2 = b.reshape(VOCAB, 1)
    out_t = pl.pallas_call(
        _tc_body,
        grid=(2, NVT),
        in_specs=[
            pl.BlockSpec((VB, DIM), lambda p, v: (v, 0)),
            pl.BlockSpec((VB, 1), lambda p, v: (v, 0)),
            pl.BlockSpec((DIM, BATCH), lambda p, v: (0, 0)),
        ],
        out_specs=pl.BlockSpec((VB, BATCH), lambda p, v: (v * p, 0)),
        out_shape=jax.ShapeDtypeStruct((VOCAB, BATCH), jnp.float32),
        scratch_shapes=[
            pltpu.VMEM((1, BATCH), jnp.float32),
        ],
        compiler_params=pltpu.CompilerParams(
            dimension_semantics=("arbitrary", "arbitrary"),
            vmem_limit_bytes=66_000_000,
        ),
    )(wt16, b2, avgt16)
    return out_t.T


# speedup vs baseline: 1.3593x; 1.2309x over previous
import functools

import jax
import jax.numpy as jnp
from jax import lax
from jax.experimental import pallas as pl
from jax.experimental.pallas import tpu as pltpu
from jax.experimental.pallas import tpu_sc as plsc

VOCAB = 100000
DIM = 128
BATCH = 4096
CTX = 20

NC = 2
NS = 16
NW = NC * NS
B_PER_W = BATCH // NW
CH = 32
NCH = B_PER_W // CH
LANES = 16

VB = 1000
NVT = VOCAB // VB


def _sc_avg_body(idx_hbm, table_hbm, out_hbm, idx_v, rows_v, acc_v, sem):
    wid = lax.axis_index("s") * NC + lax.axis_index("c")
    base = wid * B_PER_W

    def chunk_body(ch, carry):
        pltpu.sync_copy(idx_hbm.at[wid, ch], idx_v)
        handles = [
            pltpu.async_copy(table_hbm.at[idx_v.at[t]], rows_v.at[t], sem)
            for t in range(CTX)
        ]
        for h in handles:
            h.wait()

        def row_body(r, c2):
            for d in range(DIM // LANES):
                sl = pl.ds(d * LANES, LANES)
                a = rows_v[0, r, sl]
                for t in range(1, CTX):
                    a = a + rows_v[t, r, sl]
                acc_v[r, sl] = a * (1.0 / CTX)
            return c2

        lax.fori_loop(0, CH, row_body, 0)
        pltpu.sync_copy(acc_v, out_hbm.at[pl.ds(base + ch * CH, CH)])
        return carry

    lax.fori_loop(0, NCH, chunk_body, 0)


_sc_avg = functools.partial(
    pl.kernel,
    mesh=plsc.VectorSubcoreMesh(core_axis_name="c", subcore_axis_name="s"),
    out_type=jax.ShapeDtypeStruct((BATCH, DIM), jnp.float32),
    scratch_types=[
        pltpu.VMEM((CTX, CH), jnp.int32),
        pltpu.VMEM((CTX, CH, DIM), jnp.float32),
        pltpu.VMEM((CH, DIM), jnp.float32),
        pltpu.SemaphoreType.DMA,
    ],
)(_sc_avg_body)


HB = BATCH // 2


def _tc_body(wt_ref, b_ref, avgt_ref, out_ref, s_ref):
    s = pl.program_id(0)
    v = pl.program_id(1)

    @pl.when((s == 0) & (v == 0))
    def _init():
        s_ref[...] = jnp.zeros_like(s_ref)

    def _acc(lo):
        raw = lax.dot_general(
            wt_ref[...], avgt_ref[:, lo:lo + HB], (((1,), (0,)), ((), ())),
            preferred_element_type=jnp.float32,
        )
        e16 = jnp.exp(raw).astype(jnp.bfloat16)
        eb = jnp.exp(b_ref[...]).astype(jnp.bfloat16)
        s_ref[:, lo:lo + HB] += lax.dot_general(
            eb, e16, (((0,), (0,)), ((), ())),
            preferred_element_type=jnp.float32,
        )

    def _write(lo):
        raw = lax.dot_general(
            wt_ref[...], avgt_ref[:, lo:lo + HB], (((1,), (0,)), ((), ())),
            preferred_element_type=jnp.float32,
        )
        out_ref[...] = (raw + b_ref[...]) - s_ref[:, lo:lo + HB]

    @pl.when(s == 0)
    def _acc_a():
        _acc(0)

    @pl.when(s == 1)
    def _mid():
        @pl.when(v == 0)
        def _fin_a():
            s_ref[:, :HB] = jnp.log(s_ref[:, :HB])
        _acc(HB)
        _write(0)

    @pl.when(s == 2)
    def _tail():
        @pl.when(v == 0)
        def _fin_b():
            s_ref[:, HB:] = jnp.log(s_ref[:, HB:])
        _write(HB)


def kernel(context_words, emb_table, W, b):
    idx4 = (
        context_words.astype(jnp.int32)
        .reshape(NW, NCH, CH, CTX)
        .transpose(0, 1, 3, 2)
    )
    avg = _sc_avg(idx4, emb_table)

    wt16 = W.T.astype(jnp.bfloat16)
    avgt16 = avg.T.astype(jnp.bfloat16)
    b2 = b.reshape(VOCAB, 1)
    out_t = pl.pallas_call(
        _tc_body,
        grid=(3, NVT),
        in_specs=[
            pl.BlockSpec((VB, DIM), lambda s, v: (v, 0)),
            pl.BlockSpec((VB, 1), lambda s, v: (v, 0)),
            pl.BlockSpec((DIM, BATCH), lambda s, v: (0, 0)),
        ],
        out_specs=pl.BlockSpec(
            (VB, HB),
            lambda s, v: (jnp.where(s == 0, 0, v), jnp.maximum(s - 1, 0)),
        ),
        out_shape=jax.ShapeDtypeStruct((VOCAB, BATCH), jnp.float32),
        scratch_shapes=[
            pltpu.VMEM((1, BATCH), jnp.float32),
        ],
        compiler_params=pltpu.CompilerParams(
            dimension_semantics=("arbitrary", "arbitrary"),
            vmem_limit_bytes=66_000_000,
        ),
    )(wt16, b2, avgt16)
    return out_t.T
